# baseline (speedup 1.0000x reference)
import functools

import jax
import jax.numpy as jnp
from jax import lax
from jax.experimental import pallas as pl
from jax.experimental.pallas import tpu as pltpu
from jax.experimental.pallas import tpu_sc as plsc

_ROWS = 4096
_COLS = 4096
_NNZ = 500000


def _build(rows, cols, nnz, *, num_cores=2, num_subcores=16, chunk=2048,
           win_cells=32768, interpret=False):
  nw = num_cores * num_subcores
  out_cells = rows * cols
  region = out_cells // nw
  nwin = region // win_cells
  assert nwin * win_cells == region and nwin >= 3
  assert nnz >= chunk and (nnz - chunk) % 8 == 0
  nvreg = chunk // 16
  bcols = nwin + 1 + (-(nwin + 1)) % 8
  assert bcols <= 24

  mesh = plsc.VectorSubcoreMesh(core_axis_name="c", subcore_axis_name="s",
                                num_cores=num_cores,
                                num_subcores=num_subcores)

  @functools.partial(
      pl.kernel,
      out_type=jax.ShapeDtypeStruct((out_cells,), jnp.float32),
      mesh=mesh,
      compiler_params=pltpu.CompilerParams(needs_layout_passes=False),
      scratch_types=[
          pltpu.VMEM((chunk,), jnp.int32),
          pltpu.VMEM((chunk,), jnp.int32),
          pltpu.VMEM((chunk,), jnp.int32),
          pltpu.VMEM((chunk,), jnp.int32),
          pltpu.VMEM((chunk,), jnp.int32),
          pltpu.VMEM((chunk,), jnp.float32),
          pltpu.VMEM((chunk,), jnp.float32),
          pltpu.VMEM((chunk,), jnp.float32),
          pltpu.VMEM((chunk,), jnp.float32),
          pltpu.VMEM((chunk,), jnp.float32),
          pltpu.VMEM((win_cells,), jnp.float32),
          pltpu.VMEM((win_cells,), jnp.float32),
          pltpu.VMEM((win_cells,), jnp.float32),
          pltpu.VMEM((bcols,), jnp.int32),
          pltpu.SemaphoreType.DMA,
          pltpu.SemaphoreType.DMA,
          pltpu.SemaphoreType.DMA,
          pltpu.SemaphoreType.DMA,
          pltpu.SemaphoreType.DMA,
          pltpu.SemaphoreType.DMA,
          pltpu.SemaphoreType.DMA,
          pltpu.SemaphoreType.DMA,
          pltpu.SemaphoreType.DMA,
          pltpu.SemaphoreType.DMA,
          pltpu.SemaphoreType.DMA,
          pltpu.SemaphoreType.DMA,
          pltpu.SemaphoreType.DMA,
      ],
      interpret=interpret,
  )
  def sc_scatter(lin_hbm, val_hbm, bounds_hbm, out_hbm,
                 lin_b0, lin_b1, lin_b2, lin_b3, lin_b4,
                 val_b0, val_b1, val_b2, val_b3, val_b4,
                 win_b0, win_b1, win_b2, brow,
                 sem_l0, sem_l1, sem_l2, sem_l3, sem_l4,
                 sem_v0, sem_v1, sem_v2, sem_v3, sem_v4,
                 sem_o0, sem_o1, sem_o2):
    wid = lax.axis_index("s") * num_cores + lax.axis_index("c")
    lane = lax.iota(jnp.int32, 16)
    base_cell = wid * region

    lin_bufs = (lin_b0, lin_b1, lin_b2, lin_b3, lin_b4)
    val_bufs = (val_b0, val_b1, val_b2, val_b3, val_b4)
    win_bufs = (win_b0, win_b1, win_b2)
    sems_l = (sem_l0, sem_l1, sem_l2, sem_l3, sem_l4)
    sems_v = (sem_v0, sem_v1, sem_v2, sem_v3, sem_v4)
    sems_o = (sem_o0, sem_o1, sem_o2)

    pltpu.sync_copy(bounds_hbm.at[wid], brow)
    b0 = brow[pl.ds(0, 16)]
    b1 = brow[pl.ds(8, 16)]

    def bval(i):
      if i < 16:
        return jnp.max(jnp.where(lane == i, b0, 0))
      return jnp.max(jnp.where(lane == (i - 8), b1, 0))

    bvals = [bval(i) for i in range(nwin + 1)]
    starts = [jnp.minimum(bvals[v] & -8, nnz - chunk) for v in range(nwin)]
    nchs = [(bvals[v + 1] - starts[v] + chunk - 1) // chunk
            for v in range(nwin)]

    def start_chunk(off, p):
      off = pl.multiple_of(off, 8)
      pltpu.async_copy(lin_hbm.at[pl.ds(off, chunk)], lin_bufs[p], sems_l[p])
      pltpu.async_copy(val_hbm.at[pl.ds(off, chunk)], val_bufs[p], sems_v[p])

    def wait_chunk(p):
      pltpu.make_async_copy(lin_hbm.at[pl.ds(0, chunk)],
                            lin_bufs[p], sems_l[p]).wait()
      pltpu.make_async_copy(val_hbm.at[pl.ds(0, chunk)],
                            val_bufs[p], sems_v[p]).wait()

    def wait_out(p):
      pltpu.make_async_copy(win_bufs[p],
                            out_hbm.at[pl.ds(0, win_cells)], sems_o[p]).wait()

    zero = jnp.zeros((16,), jnp.float32)
    ucap = jnp.uint32(win_cells)

    def replay(lb, vb, wb, wbase):
      @pl.loop(0, nvreg, unroll=8)
      def _r(j):
        q = lb[pl.ds(j * 16, 16)]
        v = vb[pl.ds(j * 16, 16)]
        local = q - wbase
        m = plsc.bitcast(local, jnp.uint32) < ucap
        plsc.store_scatter(wb, [local], v, mask=m)

    def unscatter(lb, wb, wbase):
      @pl.loop(0, nvreg, unroll=8)
      def _u(j):
        q = lb[pl.ds(j * 16, 16)]
        local = q - wbase
        m = plsc.bitcast(local, jnp.uint32) < ucap
        plsc.store_scatter(wb, [local], zero, mask=m)

    start_chunk(starts[0], 0)
    start_chunk(starts[1], 1)

    for win in range(nwin):
      p = win % 3
      s = win % 5
      wb = win_bufs[p]
      wbase = base_cell + win * win_cells

      if win >= 3:
        wait_out(p)
        sp = (win - 3) % 5

        @pl.when(nchs[win - 3] == 1)
        def _fast():
          unscatter(lin_bufs[sp], wb, base_cell + (win - 3) * win_cells)

        @pl.when(nchs[win - 3] != 1)
        def _slow():
          @pl.loop(0, win_cells // 16, unroll=8)
          def _zero(z):
            wb[pl.ds(z * 16, 16)] = zero
      else:
        @pl.loop(0, win_cells // 16, unroll=8)
        def _zero(z):
          wb[pl.ds(z * 16, 16)] = zero

      wait_chunk(s)
      replay(lin_bufs[s], val_bufs[s], wb, wbase)

      @pl.loop(1, nchs[win])
      def _extra(k):
        off = pl.multiple_of(
            jnp.minimum(starts[win] + k * chunk, nnz - chunk), 8)
        pltpu.sync_copy(lin_hbm.at[pl.ds(off, chunk)], lin_bufs[s])
        pltpu.sync_copy(val_hbm.at[pl.ds(off, chunk)], val_bufs[s])
        replay(lin_bufs[s], val_bufs[s], wb, wbase)

      pltpu.async_copy(
          wb, out_hbm.at[pl.ds(pl.multiple_of(wbase, 8), win_cells)],
          sems_o[p])
      if win + 2 < nwin:
        start_chunk(starts[win + 2], (win + 2) % 5)

    wait_out(nwin % 3)
    wait_out((nwin + 1) % 3)
    wait_out((nwin + 2) % 3)

  return sc_scatter, nw * nwin


@functools.cache
def _built():
  return _build(_ROWS, _COLS, _NNZ)


def kernel(gt_index, gt_loc, dest_shape):
  del dest_shape
  fn, nwin_total = _built()
  lin = gt_index[:, 0] * 4096 + gt_index[:, 1]
  slin, sval = lax.sort((lin, gt_loc), dimension=0, num_keys=1,
                        is_stable=False)
  win_cells = (_ROWS * _COLS) // nwin_total
  edges = jnp.arange(nwin_total + 1, dtype=jnp.int32) * win_cells
  b = jnp.searchsorted(slin, edges, side="left").astype(jnp.int32)
  nwin = nwin_total // 32
  bcols = nwin + 1 + (-(nwin + 1)) % 8
  bidx = jnp.minimum(
      jnp.arange(32, dtype=jnp.int32)[:, None] * nwin
      + jnp.arange(bcols, dtype=jnp.int32)[None, :], nwin_total)
  bounds = b[bidx]
  out = fn(slin, sval, bounds)
  return out.reshape(_ROWS, _COLS, 1)


# speedup vs baseline: 1.0005x; 1.0005x over previous
import functools

import jax
import jax.numpy as jnp
from jax import lax
from jax.experimental import pallas as pl
from jax.experimental.pallas import tpu as pltpu
from jax.experimental.pallas import tpu_sc as plsc

_ROWS = 4096
_COLS = 4096
_NNZ = 500000


def _build(rows, cols, nnz, *, num_cores=2, num_subcores=16, chunk=2048,
           win_cells=32768, interpret=False):
  nw = num_cores * num_subcores
  out_cells = rows * cols
  region = out_cells // nw
  nwin = region // win_cells
  assert nwin * win_cells == region and nwin >= 2
  assert nnz >= chunk and (nnz - chunk) % 8 == 0
  nvreg = chunk // 16
  bcols = nwin + 1 + (-(nwin + 1)) % 8
  assert bcols <= 24

  mesh = plsc.VectorSubcoreMesh(core_axis_name="c", subcore_axis_name="s",
                                num_cores=num_cores,
                                num_subcores=num_subcores)

  @functools.partial(
      pl.kernel,
      out_type=jax.ShapeDtypeStruct((out_cells,), jnp.float32),
      mesh=mesh,
      compiler_params=pltpu.CompilerParams(needs_layout_passes=False),
      scratch_types=[
          pltpu.VMEM((chunk,), jnp.int32),
          pltpu.VMEM((chunk,), jnp.int32),
          pltpu.VMEM((chunk,), jnp.int32),
          pltpu.VMEM((chunk,), jnp.int32),
          pltpu.VMEM((chunk,), jnp.float32),
          pltpu.VMEM((chunk,), jnp.float32),
          pltpu.VMEM((chunk,), jnp.float32),
          pltpu.VMEM((chunk,), jnp.float32),
          pltpu.VMEM((win_cells,), jnp.float32),
          pltpu.VMEM((win_cells,), jnp.float32),
          pltpu.VMEM((bcols,), jnp.int32),
          pltpu.SemaphoreType.DMA,
          pltpu.SemaphoreType.DMA,
          pltpu.SemaphoreType.DMA,
          pltpu.SemaphoreType.DMA,
          pltpu.SemaphoreType.DMA,
          pltpu.SemaphoreType.DMA,
          pltpu.SemaphoreType.DMA,
          pltpu.SemaphoreType.DMA,
          pltpu.SemaphoreType.DMA,
          pltpu.SemaphoreType.DMA,
      ],
      interpret=interpret,
  )
  def sc_scatter(lin_hbm, val_hbm, bounds_hbm, out_hbm,
                 lin_b0, lin_b1, lin_b2, lin_b3,
                 val_b0, val_b1, val_b2, val_b3, win_b0, win_b1, brow,
                 sem_l0, sem_l1, sem_l2, sem_l3,
                 sem_v0, sem_v1, sem_v2, sem_v3, sem_o0, sem_o1):
    wid = lax.axis_index("s") * num_cores + lax.axis_index("c")
    lane = lax.iota(jnp.int32, 16)
    base_cell = wid * region

    lin_bufs = (lin_b0, lin_b1, lin_b2, lin_b3)
    val_bufs = (val_b0, val_b1, val_b2, val_b3)
    win_bufs = (win_b0, win_b1)
    sems_l = (sem_l0, sem_l1, sem_l2, sem_l3)
    sems_v = (sem_v0, sem_v1, sem_v2, sem_v3)
    sems_o = (sem_o0, sem_o1)

    pltpu.sync_copy(bounds_hbm.at[wid], brow)
    b0 = brow[pl.ds(0, 16)]
    b1 = brow[pl.ds(8, 16)]

    def bval(i):
      if i < 16:
        return jnp.max(jnp.where(lane == i, b0, 0))
      return jnp.max(jnp.where(lane == (i - 8), b1, 0))

    bvals = [bval(i) for i in range(nwin + 1)]
    starts = [jnp.minimum(bvals[v] & -8, nnz - chunk) for v in range(nwin)]
    nchs = [(bvals[v + 1] - starts[v] + chunk - 1) // chunk
            for v in range(nwin)]

    def start_chunk(off, p):
      off = pl.multiple_of(off, 8)
      pltpu.async_copy(lin_hbm.at[pl.ds(off, chunk)], lin_bufs[p], sems_l[p])
      pltpu.async_copy(val_hbm.at[pl.ds(off, chunk)], val_bufs[p], sems_v[p])

    def wait_chunk(p):
      pltpu.make_async_copy(lin_hbm.at[pl.ds(0, chunk)],
                            lin_bufs[p], sems_l[p]).wait()
      pltpu.make_async_copy(val_hbm.at[pl.ds(0, chunk)],
                            val_bufs[p], sems_v[p]).wait()

    def wait_out(p):
      pltpu.make_async_copy(win_bufs[p],
                            out_hbm.at[pl.ds(0, win_cells)], sems_o[p]).wait()

    zero = jnp.zeros((16,), jnp.float32)
    ucap = jnp.uint32(win_cells)

    def replay(lb, vb, wb, wbase):
      @pl.loop(0, nvreg, unroll=8)
      def _r(j):
        q = lb[pl.ds(j * 16, 16)]
        v = vb[pl.ds(j * 16, 16)]
        local = q - wbase
        m = plsc.bitcast(local, jnp.uint32) < ucap
        plsc.store_scatter(wb, [local], v, mask=m)

    def unscatter(lb, wb, wbase):
      @pl.loop(0, nvreg, unroll=8)
      def _u(j):
        q = lb[pl.ds(j * 16, 16)]
        local = q - wbase
        m = plsc.bitcast(local, jnp.uint32) < ucap
        plsc.store_scatter(wb, [local], zero, mask=m)

    start_chunk(starts[0], 0)
    start_chunk(starts[1], 1)

    for win in range(nwin):
      p = win % 2
      s = win % 4
      wb = win_bufs[p]
      wbase = base_cell + win * win_cells

      if win >= 2:
        wait_out(p)
        sp = (win - 2) % 4

        @pl.when(nchs[win - 2] == 1)
        def _fast():
          unscatter(lin_bufs[sp], wb, base_cell + (win - 2) * win_cells)

        @pl.when(nchs[win - 2] != 1)
        def _slow():
          @pl.loop(0, win_cells // 16, unroll=8)
          def _zero(z):
            wb[pl.ds(z * 16, 16)] = zero
      else:
        @pl.loop(0, win_cells // 16, unroll=8)
        def _zero(z):
          wb[pl.ds(z * 16, 16)] = zero

      wait_chunk(s)
      replay(lin_bufs[s], val_bufs[s], wb, wbase)

      @pl.loop(1, nchs[win])
      def _extra(k):
        off = pl.multiple_of(
            jnp.minimum(starts[win] + k * chunk, nnz - chunk), 8)
        pltpu.sync_copy(lin_hbm.at[pl.ds(off, chunk)], lin_bufs[s])
        pltpu.sync_copy(val_hbm.at[pl.ds(off, chunk)], val_bufs[s])
        replay(lin_bufs[s], val_bufs[s], wb, wbase)

      pltpu.async_copy(
          wb, out_hbm.at[pl.ds(pl.multiple_of(wbase, 8), win_cells)],
          sems_o[p])
      if win + 2 < nwin:
        start_chunk(starts[win + 2], (win + 2) % 4)

    wait_out(nwin % 2)
    wait_out((nwin + 1) % 2)

  return sc_scatter, nw * nwin


@functools.cache
def _built():
  return _build(_ROWS, _COLS, _NNZ)


def kernel(gt_index, gt_loc, dest_shape):
  del dest_shape
  fn, nwin_total = _built()
  lin = gt_index[:, 0] * 4096 + gt_index[:, 1]
  slin, sval = lax.sort((lin, gt_loc), dimension=0, num_keys=1,
                        is_stable=False)
  win_cells = (_ROWS * _COLS) // nwin_total
  edges = jnp.arange(nwin_total + 1, dtype=jnp.int32) * win_cells
  b = jnp.searchsorted(slin, edges, side="left").astype(jnp.int32)
  nwin = nwin_total // 32
  bcols = nwin + 1 + (-(nwin + 1)) % 8
  bidx = jnp.minimum(
      jnp.arange(32, dtype=jnp.int32)[:, None] * nwin
      + jnp.arange(bcols, dtype=jnp.int32)[None, :], nwin_total)
  bounds = b[bidx]
  out = fn(slin, sval, bounds)
  return out.reshape(_ROWS, _COLS, 1)
